# Initial kernel scaffold; baseline (speedup 1.0000x reference)
import jax
import jax.numpy as jnp
from jax.experimental import pallas as pl


def kernel(inputs, table):
    raise NotImplementedError("write your pallas kernel here")


# speedup vs baseline: 1.4768x; 1.4768x over previous
import functools

import jax
import jax.numpy as jnp
from jax import lax
from jax.experimental import pallas as pl
from jax.experimental.pallas import tpu as pltpu
from jax.experimental.pallas import tpu_sc as plsc

_BATCH, _HIST, _DIM = 4096, 200, 32
_B = _BATCH * _HIST

_info = plsc.get_sparse_core_info()
_NC, _NS = _info.num_cores, _info.num_subcores
_NW = _NC * _NS
_BPW = _B // _NW
_CH = 1600
_NCHUNK = _BPW // _CH

_mesh = plsc.VectorSubcoreMesh(core_axis_name="c", subcore_axis_name="s")


@functools.partial(
    pl.kernel,
    mesh=_mesh,
    out_type=jax.ShapeDtypeStruct((_B, _DIM), jnp.float32),
    scratch_types=[
        pltpu.VMEM((_CH,), jnp.int32),
        pltpu.VMEM((_CH, _DIM), jnp.float32),
        pltpu.SemaphoreType.DMA,
    ],
    compiler_params=pltpu.CompilerParams(use_tc_tiling_on_sc=False),
)
def _gather(idx_hbm, table_hbm, out_hbm, idx_v, rows_v, sem):
    wid = lax.axis_index("s") * _NC + lax.axis_index("c")
    base = wid * _BPW

    def body(i, carry):
        off = base + i * _CH
        pltpu.sync_copy(idx_hbm.at[pl.ds(off, _CH)], idx_v)
        pltpu.async_copy(table_hbm.at[idx_v], rows_v, sem).wait()
        pltpu.sync_copy(rows_v, out_hbm.at[pl.ds(off, _CH)])
        return carry

    lax.fori_loop(0, _NCHUNK, body, 0)


def kernel(inputs, table):
    flat = inputs.reshape(-1).astype(jnp.int32)
    out = _gather(flat, table)
    return out.reshape(_BATCH, _HIST, _DIM)


# speedup vs baseline: 1.4910x; 1.0096x over previous
import functools

import jax
import jax.numpy as jnp
from jax import lax
from jax.experimental import pallas as pl
from jax.experimental.pallas import tpu as pltpu
from jax.experimental.pallas import tpu_sc as plsc

_BATCH, _HIST, _DIM = 4096, 200, 32
_B = _BATCH * _HIST

_info = plsc.get_sparse_core_info()
_NC, _NS = _info.num_cores, _info.num_subcores
_NW = _NC * _NS
_BPW = _B // _NW
_CH = 1600
_NCHUNK = _BPW // _CH

_mesh = plsc.VectorSubcoreMesh(core_axis_name="c", subcore_axis_name="s")


@functools.partial(
    pl.kernel,
    mesh=_mesh,
    out_type=jax.ShapeDtypeStruct((_B, _DIM), jnp.float32),
    scratch_types=[
        pltpu.VMEM((_CH,), jnp.int32),
        pltpu.VMEM((_CH,), jnp.int32),
        pltpu.VMEM((_CH, _DIM), jnp.float32),
        pltpu.VMEM((_CH, _DIM), jnp.float32),
        pltpu.SemaphoreType.DMA,
        pltpu.SemaphoreType.DMA,
    ],
    compiler_params=pltpu.CompilerParams(use_tc_tiling_on_sc=False),
)
def _gather(idx_hbm, table_hbm, out_hbm, idx0, idx1, rows0, rows1, gat_sem, out_sem):
    wid = lax.axis_index("s") * _NC + lax.axis_index("c")
    base = wid * _BPW
    idx_v = [idx0, idx1]
    rows_v = [rows0, rows1]

    def load_idx(i, b):
        pltpu.sync_copy(idx_hbm.at[pl.ds(base + i * _CH, _CH)], idx_v[b])

    def start_gather(b):
        return pltpu.async_copy(table_hbm.at[idx_v[b]], rows_v[b], gat_sem)

    def start_out(i, b):
        return pltpu.async_copy(
            rows_v[b], out_hbm.at[pl.ds(base + i * _CH, _CH)], out_sem
        )

    load_idx(0, 0)
    gathers = [start_gather(0)]
    outs = []
    for i in range(_NCHUNK):
        b = i % 2
        if i + 1 < _NCHUNK:
            load_idx(i + 1, 1 - b)
        gathers[i].wait()
        if i >= 1:
            outs[i - 1].wait()
        if i + 1 < _NCHUNK:
            gathers.append(start_gather(1 - b))
        outs.append(start_out(i, b))
    outs[-1].wait()


def kernel(inputs, table):
    flat = inputs.reshape(-1).astype(jnp.int32)
    out = _gather(flat, table)
    return out.reshape(_BATCH, _HIST, _DIM)
